# baseline (device time: 65611 ns/iter reference)
import jax
import jax.numpy as jnp
from jax import lax
from jax.experimental import pallas as pl
from jax.experimental.pallas import tpu as pltpu

M, N = 1024, 1024

A1_OPS = (("h", 0), ("h", 1), ("d", 3), ("d", 2), ("g", 1), ("g", 0))
A2_OPS = (("h", 1), ("h", 0), ("d", 2), ("d", 3), ("g", 0), ("g", 1))
B1_OPS = (("h", 3), ("h", 2), ("d", 0), ("d", 1), ("g", 2), ("g", 3))
B2_OPS = (("h", 2), ("h", 3), ("d", 1), ("d", 0), ("g", 3), ("g", 2))

CHUNKS = [
    (896, 64, B1_OPS),
    (960, 64, B2_OPS),
    (0, 128, A1_OPS),
    (128, 128, A1_OPS),
    (256, 128, A1_OPS),
    (384, 64, A1_OPS),
    (448, 128, A2_OPS),
    (576, 128, A2_OPS),
    (704, 128, A2_OPS),
    (832, 64, A2_OPS),
]
NCHUNK = len(CHUNKS)
NROUND = 6


def _round_sizes(rows, ops):
    sizes = []
    ln = rows
    for kind, _ in ops:
        if kind == "h":
            sizes.append(ln // 2)
            ln //= 2
        elif kind == "d":
            sizes.append(ln)
        else:
            sizes.append(ln)
            ln *= 2
    return sizes


SIZES = [_round_sizes(rows, ops) for _, rows, ops in CHUNKS]
SCRATCH_IDX = {}
_scratch_shapes = []
for _c, (_b, _rows, _ops) in enumerate(CHUNKS):
    for _r, (_kind, _d) in enumerate(_ops):
        if _kind in ("h", "d"):
            SCRATCH_IDX[(_c, _r)] = len(_scratch_shapes)
            _scratch_shapes.append((SIZES[_c][_r], N))


def kernel(x):
    x2 = x.reshape(M, N)

    def body(x_ref, out_ref, *scratch):
        rbufs = scratch[: len(_scratch_shapes)]
        send_sems = scratch[-2]
        recv_sems = scratch[-1]

        mx = lax.axis_index("x")
        my = lax.axis_index("y")
        mz = lax.axis_index("z")

        bits = [mx, my, mz // 2, mz % 2]
        partners = [
            (1 - mx, my, mz),
            (mx, 1 - my, mz),
            (mx, my, jnp.bitwise_xor(mz, 2)),
            (mx, my, jnp.bitwise_xor(mz, 1)),
        ]

        barrier_sem = pltpu.get_barrier_semaphore()
        for d in range(4):
            pl.semaphore_signal(
                barrier_sem,
                inc=1,
                device_id=partners[d],
                device_id_type=pl.DeviceIdType.MESH,
            )
        pl.semaphore_wait(barrier_sem, 4)

        out_ref[:, :] = x_ref[:, :]

        base = [jnp.int32(cb) for cb, _, _ in CHUNKS]

        def mk(c, r):
            kind, d = CHUNKS[c][2][r]
            sz = SIZES[c][r]
            if kind == "h":
                src_base = base[c] + (1 - bits[d]) * sz
                dst = rbufs[SCRATCH_IDX[(c, r)]]
            elif kind == "d":
                src_base = base[c]
                dst = rbufs[SCRATCH_IDX[(c, r)]]
            else:
                src_base = base[c]
                dst = out_ref.at[pl.ds(base[c], sz)]
            return pltpu.make_async_remote_copy(
                src_ref=out_ref.at[pl.ds(src_base, sz)],
                dst_ref=dst,
                send_sem=send_sems.at[c * NROUND + r],
                recv_sem=recv_sems.at[c * NROUND + r],
                device_id=partners[d],
                device_id_type=pl.DeviceIdType.MESH,
            )

        pend = {}
        for c in range(NCHUNK):
            pend[c] = mk(c, 0)
            pend[c].start()
        for r in range(NROUND):
            for c in range(NCHUNK):
                kind, d = CHUNKS[c][2][r]
                sz = SIZES[c][r]
                pend[c].wait()
                if kind == "h":
                    keep = base[c] + bits[d] * sz
                    out_ref[pl.ds(keep, sz), :] = (
                        out_ref[pl.ds(keep, sz), :]
                        + rbufs[SCRATCH_IDX[(c, r)]][:, :]
                    )
                    base[c] = keep
                elif kind == "d":
                    out_ref[pl.ds(base[c], sz), :] = (
                        out_ref[pl.ds(base[c], sz), :]
                        + rbufs[SCRATCH_IDX[(c, r)]][:, :]
                    )
                else:
                    base[c] = base[c] - bits[d] * sz
                if r + 1 < NROUND:
                    pend[c] = mk(c, r + 1)
                    pend[c].start()

    return pl.pallas_call(
        body,
        out_shape=jax.ShapeDtypeStruct((M, N), jnp.float32),
        in_specs=[pl.BlockSpec(memory_space=pltpu.VMEM)],
        out_specs=pl.BlockSpec(memory_space=pltpu.VMEM),
        scratch_shapes=[
            pltpu.VMEM(shape, jnp.float32) for shape in _scratch_shapes
        ]
        + [
            pltpu.SemaphoreType.DMA((NCHUNK * NROUND,)),
            pltpu.SemaphoreType.DMA((NCHUNK * NROUND,)),
        ],
        compiler_params=pltpu.CompilerParams(collective_id=0),
    )(x2)


# device time: 57058 ns/iter; 1.1499x vs baseline; 1.1499x over previous
import jax
import jax.numpy as jnp
from jax import lax
from jax.experimental import pallas as pl
from jax.experimental.pallas import tpu as pltpu

M, N = 1024, 1024

A1_OPS = (("h", 0), ("h", 1), ("d", 3), ("d", 2), ("g", 1), ("g", 0))
A2_OPS = (("h", 1), ("h", 0), ("d", 2), ("d", 3), ("g", 0), ("g", 1))
B1_OPS = (("h", 3), ("h", 2), ("d", 0), ("d", 1), ("g", 2), ("g", 3))
B2_OPS = (("h", 2), ("h", 3), ("d", 1), ("d", 0), ("g", 3), ("g", 2))

CHUNKS = [
    (576, 224, B1_OPS),
    (800, 224, B2_OPS),
    (0, 288, A1_OPS),
    (288, 288, A2_OPS),
]
NCHUNK = len(CHUNKS)
NROUND = 6


def _round_sizes(rows, ops):
    sizes = []
    ln = rows
    for kind, _ in ops:
        if kind == "h":
            sizes.append(ln // 2)
            ln //= 2
        elif kind == "d":
            sizes.append(ln)
        else:
            sizes.append(ln)
            ln *= 2
    return sizes


SIZES = [_round_sizes(rows, ops) for _, rows, ops in CHUNKS]
SCRATCH_IDX = {}
_scratch_shapes = []
for _c, (_b, _rows, _ops) in enumerate(CHUNKS):
    for _r, (_kind, _d) in enumerate(_ops):
        if _kind in ("h", "d"):
            SCRATCH_IDX[(_c, _r)] = len(_scratch_shapes)
            _scratch_shapes.append((SIZES[_c][_r], N))


def kernel(x):
    x2 = x.reshape(M, N)

    def body(x_ref, out_ref, *scratch):
        rbufs = scratch[: len(_scratch_shapes)]
        send_sems = scratch[-2]
        recv_sems = scratch[-1]

        mx = lax.axis_index("x")
        my = lax.axis_index("y")
        mz = lax.axis_index("z")

        bits = [mx, my, mz // 2, mz % 2]
        partners = [
            (1 - mx, my, mz),
            (mx, 1 - my, mz),
            (mx, my, jnp.bitwise_xor(mz, 2)),
            (mx, my, jnp.bitwise_xor(mz, 1)),
        ]

        barrier_sem = pltpu.get_barrier_semaphore()
        for d in range(4):
            pl.semaphore_signal(
                barrier_sem,
                inc=1,
                device_id=partners[d],
                device_id_type=pl.DeviceIdType.MESH,
            )
        pl.semaphore_wait(barrier_sem, 4)

        out_ref[:, :] = x_ref[:, :]

        base = [jnp.int32(cb) for cb, _, _ in CHUNKS]

        def mk(c, r):
            kind, d = CHUNKS[c][2][r]
            sz = SIZES[c][r]
            if kind == "h":
                src_base = base[c] + (1 - bits[d]) * sz
                dst = rbufs[SCRATCH_IDX[(c, r)]]
            elif kind == "d":
                src_base = base[c]
                dst = rbufs[SCRATCH_IDX[(c, r)]]
            else:
                src_base = base[c]
                dst = out_ref.at[pl.ds(base[c], sz)]
            return pltpu.make_async_remote_copy(
                src_ref=out_ref.at[pl.ds(src_base, sz)],
                dst_ref=dst,
                send_sem=send_sems.at[c * NROUND + r],
                recv_sem=recv_sems.at[c * NROUND + r],
                device_id=partners[d],
                device_id_type=pl.DeviceIdType.MESH,
            )

        pend = {}
        for c in range(NCHUNK):
            pend[c] = mk(c, 0)
            pend[c].start()
        for r in range(NROUND):
            for c in range(NCHUNK):
                kind, d = CHUNKS[c][2][r]
                sz = SIZES[c][r]
                pend[c].wait()
                if kind == "h":
                    keep = base[c] + bits[d] * sz
                    out_ref[pl.ds(keep, sz), :] = (
                        out_ref[pl.ds(keep, sz), :]
                        + rbufs[SCRATCH_IDX[(c, r)]][:, :]
                    )
                    base[c] = keep
                elif kind == "d":
                    out_ref[pl.ds(base[c], sz), :] = (
                        out_ref[pl.ds(base[c], sz), :]
                        + rbufs[SCRATCH_IDX[(c, r)]][:, :]
                    )
                else:
                    base[c] = base[c] - bits[d] * sz
                if r + 1 < NROUND:
                    pend[c] = mk(c, r + 1)
                    pend[c].start()

    return pl.pallas_call(
        body,
        out_shape=jax.ShapeDtypeStruct((M, N), jnp.float32),
        in_specs=[pl.BlockSpec(memory_space=pltpu.VMEM)],
        out_specs=pl.BlockSpec(memory_space=pltpu.VMEM),
        scratch_shapes=[
            pltpu.VMEM(shape, jnp.float32) for shape in _scratch_shapes
        ]
        + [
            pltpu.SemaphoreType.DMA((NCHUNK * NROUND,)),
            pltpu.SemaphoreType.DMA((NCHUNK * NROUND,)),
        ],
        compiler_params=pltpu.CompilerParams(collective_id=0),
    )(x2)


# device time: 55606 ns/iter; 1.1799x vs baseline; 1.0261x over previous
import jax
import jax.numpy as jnp
from jax import lax
from jax.experimental import pallas as pl
from jax.experimental.pallas import tpu as pltpu

M, N = 1024, 1024

A1_OPS = (("h", 0), ("h", 1), ("d", 3), ("d", 2), ("g", 1), ("g", 0))
A2_OPS = (("h", 1), ("h", 0), ("d", 2), ("d", 3), ("g", 0), ("g", 1))
B1_OPS = (("h", 3), ("h", 2), ("d", 0), ("d", 1), ("g", 2), ("g", 3))
B2_OPS = (("h", 2), ("h", 3), ("d", 1), ("d", 0), ("g", 3), ("g", 2))

CHUNKS = [
    (640, 192, B1_OPS),
    (832, 192, B2_OPS),
    (0, 320, A1_OPS),
    (320, 320, A2_OPS),
]
NCHUNK = len(CHUNKS)
NROUND = 6


def _round_sizes(rows, ops):
    sizes = []
    ln = rows
    for kind, _ in ops:
        if kind == "h":
            sizes.append(ln // 2)
            ln //= 2
        elif kind == "d":
            sizes.append(ln)
        else:
            sizes.append(ln)
            ln *= 2
    return sizes


SIZES = [_round_sizes(rows, ops) for _, rows, ops in CHUNKS]
SCRATCH_IDX = {}
_scratch_shapes = []
for _c, (_b, _rows, _ops) in enumerate(CHUNKS):
    for _r, (_kind, _d) in enumerate(_ops):
        if _kind in ("h", "d"):
            SCRATCH_IDX[(_c, _r)] = len(_scratch_shapes)
            _scratch_shapes.append((SIZES[_c][_r], N))


def kernel(x):
    x2 = x.reshape(M, N)

    def body(x_ref, out_ref, *scratch):
        rbufs = scratch[: len(_scratch_shapes)]
        send_sems = scratch[-2]
        recv_sems = scratch[-1]

        mx = lax.axis_index("x")
        my = lax.axis_index("y")
        mz = lax.axis_index("z")

        bits = [mx, my, mz // 2, mz % 2]
        partners = [
            (1 - mx, my, mz),
            (mx, 1 - my, mz),
            (mx, my, jnp.bitwise_xor(mz, 2)),
            (mx, my, jnp.bitwise_xor(mz, 1)),
        ]

        barrier_sem = pltpu.get_barrier_semaphore()
        for d in range(4):
            pl.semaphore_signal(
                barrier_sem,
                inc=1,
                device_id=partners[d],
                device_id_type=pl.DeviceIdType.MESH,
            )
        pl.semaphore_wait(barrier_sem, 4)

        base = [jnp.int32(cb) for cb, _, _ in CHUNKS]

        def mk(c, r):
            kind, d = CHUNKS[c][2][r]
            sz = SIZES[c][r]
            if kind == "h":
                src_base = base[c] + (1 - bits[d]) * sz
                dst = rbufs[SCRATCH_IDX[(c, r)]]
            elif kind == "d":
                src_base = base[c]
                dst = rbufs[SCRATCH_IDX[(c, r)]]
            else:
                src_base = base[c]
                dst = out_ref.at[pl.ds(base[c], sz)]
            src = x_ref if r == 0 else out_ref
            return pltpu.make_async_remote_copy(
                src_ref=src.at[pl.ds(src_base, sz)],
                dst_ref=dst,
                send_sem=send_sems.at[c * NROUND + r],
                recv_sem=recv_sems.at[c * NROUND + r],
                device_id=partners[d],
                device_id_type=pl.DeviceIdType.MESH,
            )

        pend = {}
        for c in range(NCHUNK):
            pend[c] = mk(c, 0)
            pend[c].start()
        out_ref[:, :] = x_ref[:, :]
        ROUND_ORDER = {
            0: (2, 3, 0, 1),
            1: (2, 3, 0, 1),
            2: (0, 1, 2, 3),
            3: (0, 1, 2, 3),
            4: (2, 3, 0, 1),
            5: (2, 3, 0, 1),
        }
        for r in range(NROUND):
            for c in ROUND_ORDER[r]:
                kind, d = CHUNKS[c][2][r]
                sz = SIZES[c][r]
                pend[c].wait()
                if kind == "h":
                    keep = base[c] + bits[d] * sz
                    out_ref[pl.ds(keep, sz), :] = (
                        out_ref[pl.ds(keep, sz), :]
                        + rbufs[SCRATCH_IDX[(c, r)]][:, :]
                    )
                    base[c] = keep
                elif kind == "d":
                    out_ref[pl.ds(base[c], sz), :] = (
                        out_ref[pl.ds(base[c], sz), :]
                        + rbufs[SCRATCH_IDX[(c, r)]][:, :]
                    )
                else:
                    base[c] = base[c] - bits[d] * sz
                if r + 1 < NROUND:
                    pend[c] = mk(c, r + 1)
                    pend[c].start()

    return pl.pallas_call(
        body,
        out_shape=jax.ShapeDtypeStruct((M, N), jnp.float32),
        in_specs=[pl.BlockSpec(memory_space=pltpu.VMEM)],
        out_specs=pl.BlockSpec(memory_space=pltpu.VMEM),
        scratch_shapes=[
            pltpu.VMEM(shape, jnp.float32) for shape in _scratch_shapes
        ]
        + [
            pltpu.SemaphoreType.DMA((NCHUNK * NROUND,)),
            pltpu.SemaphoreType.DMA((NCHUNK * NROUND,)),
        ],
        compiler_params=pltpu.CompilerParams(collective_id=0),
    )(x2)
